# baseline (speedup 1.0000x reference)
import jax
import jax.numpy as jnp
from jax import lax
from jax.experimental import pallas as pl
from jax.experimental.pallas import tpu as pltpu
from jax.experimental.pallas import tpu_sc as plsc

_NSEG = 512
_EPS = 1e-05
_TOT = 240
_SQW = 80
_CH = 80
_NC = 2
_NS = 16
_NW = _NC * _NS


def _sc_phase1_body(x_hbm, batch_hbm, z1_hbm, z2_hbm, p1_hbm, p2_hbm,
                    xb0, xb1, qb0, qb1, ix0, ix1,
                    si0, si1, sc0, sc1, s1_sh, sq_sh):
    c = lax.axis_index("c")
    s = lax.axis_index("s")
    wid = c * _NS + s
    rows_per_w = x_hbm.shape[0] // _NW
    nchunk = rows_per_w // _CH
    base = wid * rows_per_w

    xbufs = (xb0, xb1)
    qbufs = (qb0, qb1)
    ixs = (ix0, ix1)
    sis = (si0, si1)
    scs = (sc0, sc1)

    def issue_in(k, slot):
        off = base + k * _CH
        pltpu.async_copy(x_hbm.at[pl.ds(off, _CH)], xbufs[slot], sis[slot])
        pltpu.async_copy(batch_hbm.at[pl.ds(off, _CH)], ixs[slot], sis[slot])

    issue_in(0, 0)
    issue_in(1, 1)

    @pl.when(s == 0)
    def _():
        pltpu.sync_copy(z1_hbm, s1_sh)
        pltpu.sync_copy(z2_hbm, sq_sh)

    onec = jnp.where(lax.iota(jnp.int32, 16) == 0, 1.0, 0.0).astype(
        jnp.float32)

    def initrow(r, _):
        qb0[r, pl.ds(64, 16)] = onec
        qb1[r, pl.ds(64, 16)] = onec
        return 0

    lax.fori_loop(0, _CH, initrow, 0)
    plsc.subcore_barrier()

    def do_chunk(k, slot):
        pltpu.make_async_copy(x_hbm.at[pl.ds(base, _CH)], xbufs[slot],
                              sis[slot]).wait()
        pltpu.make_async_copy(batch_hbm.at[pl.ds(base, _CH)], ixs[slot],
                              sis[slot]).wait()

        def row(r, _):
            for csl in range(4):
                sl = pl.ds(csl * 16, 16)
                v = xbufs[slot][r, sl]
                qbufs[slot][r, sl] = v * v
            return 0

        lax.fori_loop(0, _CH, row, 0)
        pltpu.async_copy(xbufs[slot], s1_sh.at[ixs[slot]], scs[slot],
                         add=True)
        pltpu.async_copy(qbufs[slot], sq_sh.at[ixs[slot]], scs[slot],
                         add=True)
        pltpu.make_async_copy(xbufs[slot], s1_sh.at[ixs[slot]],
                              scs[slot]).wait()
        pltpu.make_async_copy(qbufs[slot], sq_sh.at[ixs[slot]],
                              scs[slot]).wait()

        @pl.when(k + 2 < nchunk)
        def _():
            issue_in(k + 2, slot)

    def pair(j, _):
        do_chunk(2 * j, 0)

        @pl.when(2 * j + 1 < nchunk)
        def _():
            do_chunk(2 * j + 1, 1)

        return 0

    lax.fori_loop(0, (nchunk + 1) // 2, pair, 0)
    plsc.subcore_barrier()

    @pl.when(s == 0)
    def _():
        pltpu.sync_copy(s1_sh, p1_hbm.at[c])
        pltpu.sync_copy(sq_sh, p2_hbm.at[c])


def _tables_body(p1_ref, p2_ref, wcol_ref, bcol_ref, a64_ref, b_ref):
    s1 = p1_ref[0] + p1_ref[1]
    sq = p2_ref[0] + p2_ref[1]
    deg = sq[:, 64:65]
    dc = jnp.maximum(deg, 1.0)
    mean = s1 / (deg + 1e-12)
    m64 = mean[:, :64]
    var = (sq[:, :64] - 2.0 * m64 * s1[:, :64] + deg * m64 * m64) / dc
    norm = jnp.sum(var, axis=1, keepdims=True) * (1.0 / 64.0)
    inv = 1.0 / (jnp.sqrt(norm) + _EPS)
    colmask = jax.lax.broadcasted_iota(jnp.int32, (_NSEG, _TOT), 1) < 64
    s_full = jnp.where(colmask, inv, 1.0)
    a = s_full * wcol_ref[...]
    a64_ref[...] = a[:, :64]
    b_ref[...] = bcol_ref[...] - mean * a


def _sc_phase2_body(x_hbm, batch_hbm, a64_hbm, b_hbm, w_hbm, o_hbm,
                    xb0, xb1, ob0, ob1, ab0, ab1, ix0, ix1, wbuf,
                    si0, si1, sg0, sg1, so0, so1, a_sh, b_sh):
    c = lax.axis_index("c")
    s = lax.axis_index("s")
    wid = c * _NS + s
    rows_per_w = x_hbm.shape[0] // _NW
    nchunk = rows_per_w // _CH
    base = wid * rows_per_w

    xbufs = (xb0, xb1)
    obufs = (ob0, ob1)
    abufs = (ab0, ab1)
    ixs = (ix0, ix1)
    sis = (si0, si1)
    sgs = (sg0, sg1)
    sos = (so0, so1)

    def issue_in(k, slot):
        off = base + k * _CH
        pltpu.async_copy(x_hbm.at[pl.ds(off, _CH)], xbufs[slot], sis[slot])
        pltpu.async_copy(batch_hbm.at[pl.ds(off, _CH)], ixs[slot], sis[slot])

    issue_in(0, 0)
    issue_in(1, 1)

    @pl.when(s == 0)
    def _():
        pltpu.sync_copy(a64_hbm, a_sh)
        pltpu.sync_copy(b_hbm, b_sh)

    pltpu.sync_copy(w_hbm, wbuf)
    wregs = [wbuf[pl.ds(64 + i * 16, 16)] for i in range(11)]
    plsc.subcore_barrier()

    def do_chunk(k, slot):
        pltpu.make_async_copy(x_hbm.at[pl.ds(base, _CH)], xbufs[slot],
                              sis[slot]).wait()
        pltpu.make_async_copy(batch_hbm.at[pl.ds(base, _CH)], ixs[slot],
                              sis[slot]).wait()

        @pl.when(k >= 2)
        def _():
            pltpu.make_async_copy(obufs[slot],
                                  o_hbm.at[pl.ds(base, _CH)],
                                  sos[slot]).wait()

        pltpu.async_copy(b_sh.at[ixs[slot]], obufs[slot], sgs[slot])
        pltpu.async_copy(a_sh.at[ixs[slot]], abufs[slot], sgs[slot])
        pltpu.make_async_copy(b_sh.at[ixs[slot]], obufs[slot],
                              sgs[slot]).wait()
        pltpu.make_async_copy(a_sh.at[ixs[slot]], abufs[slot],
                              sgs[slot]).wait()

        def row(r, _):
            for csl in range(4):
                sl = pl.ds(csl * 16, 16)
                v = xbufs[slot][r, sl] * abufs[slot][r, sl]
                plsc.addupdate(obufs[slot].at[r, sl], v)
            for i in range(11):
                sl = pl.ds(64 + i * 16, 16)
                v = xbufs[slot][r, sl] * wregs[i]
                plsc.addupdate(obufs[slot].at[r, sl], v)
            return 0

        lax.fori_loop(0, _CH, row, 0)
        off = base + k * _CH
        pltpu.async_copy(obufs[slot], o_hbm.at[pl.ds(off, _CH)], sos[slot])

        @pl.when(k + 2 < nchunk)
        def _():
            issue_in(k + 2, slot)

    def pair(j, _):
        do_chunk(2 * j, 0)

        @pl.when(2 * j + 1 < nchunk)
        def _():
            do_chunk(2 * j + 1, 1)

        return 0

    lax.fori_loop(0, (nchunk + 1) // 2, pair, 0)
    pltpu.make_async_copy(obufs[0], o_hbm.at[pl.ds(base, _CH)], sos[0]).wait()
    pltpu.make_async_copy(obufs[1], o_hbm.at[pl.ds(base, _CH)], sos[1]).wait()


def kernel(x, batch, weight, bias):
    n, tot = x.shape
    assert tot == _TOT and n % (_NW * _CH) == 0

    wcol = jnp.concatenate([
        weight[0:64],
        jnp.repeat(weight[64:96], 3),
        jnp.repeat(weight[96:112], 5),
    ])
    bcol = jnp.concatenate([bias, jnp.zeros((tot - 64,), jnp.float32)])
    z1 = jnp.zeros((_NSEG, _TOT), jnp.float32)
    z2 = jnp.zeros((_NSEG, _SQW), jnp.float32)

    mesh = plsc.VectorSubcoreMesh(core_axis_name="c", subcore_axis_name="s")

    phase1 = pl.kernel(
        _sc_phase1_body,
        out_type=(
            jax.ShapeDtypeStruct((_NC, _NSEG, _TOT), jnp.float32),
            jax.ShapeDtypeStruct((_NC, _NSEG, _SQW), jnp.float32),
        ),
        mesh=mesh,
        compiler_params=pltpu.CompilerParams(use_tc_tiling_on_sc=False),
        scratch_types=[
            pltpu.VMEM((_CH, _TOT), jnp.float32),
            pltpu.VMEM((_CH, _TOT), jnp.float32),
            pltpu.VMEM((_CH, _SQW), jnp.float32),
            pltpu.VMEM((_CH, _SQW), jnp.float32),
            pltpu.VMEM((_CH,), jnp.int32),
            pltpu.VMEM((_CH,), jnp.int32),
            pltpu.SemaphoreType.DMA,
            pltpu.SemaphoreType.DMA,
            pltpu.SemaphoreType.DMA,
            pltpu.SemaphoreType.DMA,
            pltpu.VMEM_SHARED((_NSEG, _TOT), jnp.float32),
            pltpu.VMEM_SHARED((_NSEG, _SQW), jnp.float32),
        ],
    )
    p1, p2 = phase1(x, batch, z1, z2)

    a64_tab, b_tab = pl.pallas_call(
        _tables_body,
        out_shape=[
            jax.ShapeDtypeStruct((_NSEG, 64), jnp.float32),
            jax.ShapeDtypeStruct((_NSEG, _TOT), jnp.float32),
        ],
    )(p1, p2, wcol.reshape(1, tot), bcol.reshape(1, tot))

    phase2 = pl.kernel(
        _sc_phase2_body,
        out_type=jax.ShapeDtypeStruct((n, tot), jnp.float32),
        mesh=mesh,
        compiler_params=pltpu.CompilerParams(use_tc_tiling_on_sc=False),
        scratch_types=[
            pltpu.VMEM((_CH, _TOT), jnp.float32),
            pltpu.VMEM((_CH, _TOT), jnp.float32),
            pltpu.VMEM((_CH, _TOT), jnp.float32),
            pltpu.VMEM((_CH, _TOT), jnp.float32),
            pltpu.VMEM((_CH, 64), jnp.float32),
            pltpu.VMEM((_CH, 64), jnp.float32),
            pltpu.VMEM((_CH,), jnp.int32),
            pltpu.VMEM((_CH,), jnp.int32),
            pltpu.VMEM((_TOT,), jnp.float32),
            pltpu.SemaphoreType.DMA,
            pltpu.SemaphoreType.DMA,
            pltpu.SemaphoreType.DMA,
            pltpu.SemaphoreType.DMA,
            pltpu.SemaphoreType.DMA,
            pltpu.SemaphoreType.DMA,
            pltpu.VMEM_SHARED((_NSEG, 64), jnp.float32),
            pltpu.VMEM_SHARED((_NSEG, _TOT), jnp.float32),
        ],
    )
    return phase2(x, batch, a64_tab, b_tab, wcol)
